# baseline (speedup 1.0000x reference)
import functools

import jax
import jax.numpy as jnp
from jax import lax
from jax.experimental import pallas as pl
from jax.experimental.pallas import tpu as pltpu
from jax.experimental.pallas import tpu_sc as plsc

B = 4
C = 384
HW = 128 * 128
NCLS = 21
NBINS = 32
NC = 2
NS = 16
NW = NC * NS
K = 4
L = 16

ROWS_PER_TILE = 128
PIX = ROWS_PER_TILE * 128
N_TILES = 128 // ROWS_PER_TILE

D_OFF = 0
S2_OFF = K * NBINS
CNT_OFF = 2 * K * NBINS
ACC_SIZE = 3 * K * NBINS

CHUNK_ROWS = 128 // (NW // B)


def _tc_main_body(prot_ref, feat_ref, lab_ref, d_ref, s_ref):
    x = feat_ref[0].reshape(C, PIX)
    p = prot_ref[...]
    dall = jnp.dot(p, x, preferred_element_type=jnp.float32)
    lab = lab_ref[0].reshape(1, PIX)
    cls = lax.broadcasted_iota(jnp.int32, (NBINS, PIX), 0)
    dsel = jnp.sum(jnp.where(cls == lab, dall, jnp.float32(0.0)), axis=0)
    ssq = jnp.sum(x * x, axis=0)
    d_ref[0] = dsel.reshape(ROWS_PER_TILE, 128)
    s_ref[0] = ssq.reshape(ROWS_PER_TILE, 128)


_tc_main = pl.pallas_call(
    _tc_main_body,
    grid=(B, N_TILES),
    in_specs=[
        pl.BlockSpec((NBINS, C), lambda b, j: (0, 0)),
        pl.BlockSpec((1, C, ROWS_PER_TILE, 128), lambda b, j: (b, 0, j, 0)),
        pl.BlockSpec((1, ROWS_PER_TILE, 128), lambda b, j: (b, j, 0)),
    ],
    out_specs=[
        pl.BlockSpec((1, ROWS_PER_TILE, 128), lambda b, j: (b, j, 0)),
        pl.BlockSpec((1, ROWS_PER_TILE, 128), lambda b, j: (b, j, 0)),
    ],
    out_shape=[
        jax.ShapeDtypeStruct((B, 128, 128), jnp.float32),
        jax.ShapeDtypeStruct((B, 128, 128), jnp.float32),
    ],
)


def _seg_body(d_hbm, s_hbm, lab_hbm, out_hbm,
              dv, sv, labv, acc_v, outst_v, sem0, sem1, sem2):
    wid = lax.axis_index("s") * NC + lax.axis_index("c")
    b = jnp.bitwise_and(wid, B - 1)
    ro = lax.shift_right_logical(wid, 2) * CHUNK_ROWS

    zeros = jnp.zeros((L,), jnp.float32)
    ones = jnp.full((L,), 1.0, jnp.float32)

    cd = pltpu.async_copy(d_hbm.at[b, pl.ds(ro, CHUNK_ROWS)], dv, sem0)
    cs = pltpu.async_copy(s_hbm.at[b, pl.ds(ro, CHUNK_ROWS)], sv, sem1)
    cl_ = pltpu.async_copy(lab_hbm.at[b, pl.ds(ro, CHUNK_ROWS)], labv, sem2)

    def zbody(i):
        acc_v[pl.ds(i * L, L)] = zeros
    plsc.parallel_loop(0, ACC_SIZE // L, 1, unroll=4)(zbody)

    cd.wait()
    cs.wait()
    cl_.wait()

    def pstep(j):
        r = lax.shift_right_logical(j, 3)
        cc = jnp.bitwise_and(j, 7) * L
        dd = dv[r, pl.ds(cc, L)]
        ss = sv[r, pl.ds(cc, L)]
        lab = labv[r, pl.ds(cc, L)]
        k = jnp.bitwise_and(j, K - 1)
        plsc.addupdate_scatter(acc_v, [lab + (D_OFF + k * NBINS)], dd)
        plsc.addupdate_scatter(acc_v, [lab + (S2_OFF + k * NBINS)], ss)
        plsc.addupdate_scatter(acc_v, [lab + (CNT_OFF + k * NBINS)], ones)
    plsc.parallel_loop(0, (CHUNK_ROWS * 128) // L, 1, unroll=4)(pstep)

    for q in range(3):
        for h in range(NBINS // L):
            tot = zeros
            for k in range(K):
                tot = tot + acc_v[pl.ds(q * K * NBINS + k * NBINS + h * L, L)]
            outst_v[pl.ds(q * NBINS + h * L, L)] = tot
    for h in range(NBINS // L):
        outst_v[pl.ds(3 * NBINS + h * L, L)] = zeros

    pltpu.sync_copy(outst_v, out_hbm.at[wid])


_seg_call = functools.partial(
    pl.kernel,
    out_type=jax.ShapeDtypeStruct((NW, 4 * NBINS), jnp.float32),
    mesh=plsc.VectorSubcoreMesh(core_axis_name="c", subcore_axis_name="s"),
    compiler_params=pltpu.CompilerParams(needs_layout_passes=False),
    scratch_types=[
        pltpu.VMEM((CHUNK_ROWS, 128), jnp.float32),
        pltpu.VMEM((CHUNK_ROWS, 128), jnp.float32),
        pltpu.VMEM((CHUNK_ROWS, 128), jnp.int32),
        pltpu.VMEM((ACC_SIZE,), jnp.float32),
        pltpu.VMEM((4 * NBINS,), jnp.float32),
        pltpu.SemaphoreType.DMA,
        pltpu.SemaphoreType.DMA,
        pltpu.SemaphoreType.DMA,
    ],
)(_seg_body)


def _fin_body(p_ref, prot_ref, o_ref):
    x = p_ref[...]
    s = jnp.sum(x, axis=0)
    d = s[0]
    s2 = s[1]
    cnt = s[2]
    pp = prot_ref[...]
    p2 = jnp.sum(pp * pp, axis=1)
    present = cnt > 0.0
    denom = jnp.where(present, cnt * jnp.float32(C), jnp.float32(1.0))
    term = (s2 - 2.0 * d + cnt * p2) / denom
    loss = (jnp.sum(jnp.where(present, term, jnp.float32(0.0)))
            / jnp.sum(present.astype(jnp.float32)))
    o_ref[...] = jnp.reshape(loss, (1, 1))


_fin_call = pl.pallas_call(
    _fin_body,
    out_shape=jax.ShapeDtypeStruct((1, 1), jnp.float32),
)


def kernel(features, labels, prototypes):
    protot = jnp.pad(prototypes, ((0, NBINS - NCLS), (0, 0)))
    d, s = _tc_main(protot, features, labels)
    partials = _seg_call(d, s, labels)
    loss = _fin_call(partials.reshape(NW, 4, NBINS), protot)
    return loss[0, 0]


# speedup vs baseline: 1.1353x; 1.1353x over previous
import functools

import jax
import jax.numpy as jnp
from jax import lax
from jax.experimental import pallas as pl
from jax.experimental.pallas import tpu as pltpu
from jax.experimental.pallas import tpu_sc as plsc

B = 4
C = 384
HW = 128 * 128
NCLS = 21
NBINS = 32
NC = 2
NS = 16
NW = NC * NS
K = 4
L = 16

ROWS_PER_TILE = 64
PIX = ROWS_PER_TILE * 128
N_TILES = 128 // ROWS_PER_TILE

D_OFF = 0
S2_OFF = K * NBINS
CNT_OFF = 2 * K * NBINS
ACC_SIZE = 3 * K * NBINS

CHUNK_ROWS = 128 // (NW // B)


def _tc_main_body(prot_ref, feat_ref, lab_ref, d_ref, s_ref):
    x = feat_ref[0].reshape(C, PIX)
    p = prot_ref[...]
    dall = jnp.dot(p, x, preferred_element_type=jnp.float32)
    lab = lab_ref[0].reshape(1, PIX)
    cls = lax.broadcasted_iota(jnp.int32, (NBINS, PIX), 0)
    dsel = jnp.sum(jnp.where(cls == lab, dall, jnp.float32(0.0)), axis=0)
    ssq = jnp.sum(x * x, axis=0)
    d_ref[0] = dsel.reshape(ROWS_PER_TILE, 128)
    s_ref[0] = ssq.reshape(ROWS_PER_TILE, 128)


_tc_main = pl.pallas_call(
    _tc_main_body,
    grid=(B, N_TILES),
    in_specs=[
        pl.BlockSpec((NBINS, C), lambda b, j: (0, 0)),
        pl.BlockSpec((1, C, ROWS_PER_TILE, 128), lambda b, j: (b, 0, j, 0)),
        pl.BlockSpec((1, ROWS_PER_TILE, 128), lambda b, j: (b, j, 0)),
    ],
    out_specs=[
        pl.BlockSpec((1, ROWS_PER_TILE, 128), lambda b, j: (b, j, 0)),
        pl.BlockSpec((1, ROWS_PER_TILE, 128), lambda b, j: (b, j, 0)),
    ],
    out_shape=[
        jax.ShapeDtypeStruct((B, 128, 128), jnp.float32),
        jax.ShapeDtypeStruct((B, 128, 128), jnp.float32),
    ],
)


def _seg_body(d_hbm, s_hbm, lab_hbm, out_hbm,
              dv, sv, labv, acc_v, outst_v, sem0, sem1, sem2):
    wid = lax.axis_index("s") * NC + lax.axis_index("c")
    b = jnp.bitwise_and(wid, B - 1)
    ro = lax.shift_right_logical(wid, 2) * CHUNK_ROWS

    zeros = jnp.zeros((L,), jnp.float32)
    ones = jnp.full((L,), 1.0, jnp.float32)

    cd = pltpu.async_copy(d_hbm.at[b, pl.ds(ro, CHUNK_ROWS)], dv, sem0)
    cs = pltpu.async_copy(s_hbm.at[b, pl.ds(ro, CHUNK_ROWS)], sv, sem1)
    cl_ = pltpu.async_copy(lab_hbm.at[b, pl.ds(ro, CHUNK_ROWS)], labv, sem2)

    def zbody(i):
        acc_v[pl.ds(i * L, L)] = zeros
    plsc.parallel_loop(0, ACC_SIZE // L, 1, unroll=4)(zbody)

    cd.wait()
    cs.wait()
    cl_.wait()

    def pstep(j):
        r = lax.shift_right_logical(j, 3)
        cc = jnp.bitwise_and(j, 7) * L
        dd = dv[r, pl.ds(cc, L)]
        ss = sv[r, pl.ds(cc, L)]
        lab = labv[r, pl.ds(cc, L)]
        k = jnp.bitwise_and(j, K - 1)
        plsc.addupdate_scatter(acc_v, [lab + (D_OFF + k * NBINS)], dd)
        plsc.addupdate_scatter(acc_v, [lab + (S2_OFF + k * NBINS)], ss)
        plsc.addupdate_scatter(acc_v, [lab + (CNT_OFF + k * NBINS)], ones)
    plsc.parallel_loop(0, (CHUNK_ROWS * 128) // L, 1, unroll=4)(pstep)

    for q in range(3):
        for h in range(NBINS // L):
            tot = zeros
            for k in range(K):
                tot = tot + acc_v[pl.ds(q * K * NBINS + k * NBINS + h * L, L)]
            outst_v[pl.ds(q * NBINS + h * L, L)] = tot
    for h in range(NBINS // L):
        outst_v[pl.ds(3 * NBINS + h * L, L)] = zeros

    pltpu.sync_copy(outst_v, out_hbm.at[wid])


_seg_call = functools.partial(
    pl.kernel,
    out_type=jax.ShapeDtypeStruct((NW, 4 * NBINS), jnp.float32),
    mesh=plsc.VectorSubcoreMesh(core_axis_name="c", subcore_axis_name="s"),
    compiler_params=pltpu.CompilerParams(needs_layout_passes=False),
    scratch_types=[
        pltpu.VMEM((CHUNK_ROWS, 128), jnp.float32),
        pltpu.VMEM((CHUNK_ROWS, 128), jnp.float32),
        pltpu.VMEM((CHUNK_ROWS, 128), jnp.int32),
        pltpu.VMEM((ACC_SIZE,), jnp.float32),
        pltpu.VMEM((4 * NBINS,), jnp.float32),
        pltpu.SemaphoreType.DMA,
        pltpu.SemaphoreType.DMA,
        pltpu.SemaphoreType.DMA,
    ],
)(_seg_body)


def _fin_body(p_ref, prot_ref, o_ref):
    x = p_ref[...]
    s = jnp.sum(x, axis=0)
    d = s[0]
    s2 = s[1]
    cnt = s[2]
    pp = prot_ref[...]
    p2 = jnp.sum(pp * pp, axis=1)
    present = cnt > 0.0
    denom = jnp.where(present, cnt * jnp.float32(C), jnp.float32(1.0))
    term = (s2 - 2.0 * d + cnt * p2) / denom
    loss = (jnp.sum(jnp.where(present, term, jnp.float32(0.0)))
            / jnp.sum(present.astype(jnp.float32)))
    o_ref[...] = jnp.reshape(loss, (1, 1))


_fin_call = pl.pallas_call(
    _fin_body,
    out_shape=jax.ShapeDtypeStruct((1, 1), jnp.float32),
)


def kernel(features, labels, prototypes):
    protot = jnp.pad(prototypes, ((0, NBINS - NCLS), (0, 0)))
    d, s = _tc_main(protot, features, labels)
    partials = _seg_call(d, s, labels)
    loss = _fin_call(partials.reshape(NW, 4, NBINS), protot)
    return loss[0, 0]
